# baseline (speedup 1.0000x reference)
import functools

import jax
import jax.numpy as jnp
from jax import lax
from jax.experimental import pallas as pl
from jax.experimental.pallas import tpu as pltpu
from jax.experimental.pallas import tpu_sc as plsc

E = 8
H = 768
I = 2112
T = 2048
B = 256
NB = T * 2 // B + E
SPAD = NB * B
NW = 32
TPW = T // NW


def _dispatch_kernel(x_ref, wr_ref, d1_ref, d2_ref, w1_ref, w2_ref,
                     nb_ref, bo_ref):
    x = x_ref[...]
    logits = lax.dot_general(x, wr_ref[...], (((1,), (1,)), ((), ())),
                             preferred_element_type=jnp.float32)
    lane = lax.broadcasted_iota(jnp.int32, (T, E), 1)
    m1 = jnp.max(logits, axis=1, keepdims=True)
    first1 = jnp.min(jnp.where(logits >= m1, lane, E), axis=1, keepdims=True)
    mask1 = lane == first1
    l2 = jnp.where(mask1, -jnp.inf, logits)
    m2 = jnp.max(l2, axis=1, keepdims=True)
    first2 = jnp.min(jnp.where(l2 >= m2, lane, E), axis=1, keepdims=True)
    mask2 = lane == first2
    s = jax.nn.sigmoid(m1 - m2)
    w1_ref[...] = s
    w2_ref[...] = 1.0 - s

    sel = jnp.where(mask1 | mask2, 1.0, 0.0)
    selb = sel.astype(jnp.bfloat16)

    it = lax.broadcasted_iota(jnp.int32, (B, B), 0)
    ic = lax.broadcasted_iota(jnp.int32, (B, B), 1)
    Lb = jnp.where(it > ic, 1.0, 0.0).astype(jnp.bfloat16)

    nblk = T // B
    rows = [jnp.sum(sel[i * B:(i + 1) * B, :], axis=0, keepdims=True)
            for i in range(nblk)]
    bs = jnp.concatenate(rows, axis=0)
    itb = lax.broadcasted_iota(jnp.int32, (nblk, nblk), 0)
    icb = lax.broadcasted_iota(jnp.int32, (nblk, nblk), 1)
    L8 = jnp.where(itb > icb, 1.0, 0.0)
    pref = lax.dot_general(L8, bs, (((1,), (0,)), ((), ())),
                           preferred_element_type=jnp.float32)
    counts = pref[nblk - 1:nblk, :] + bs[nblk - 1:nblk, :]
    counts_i = counts.astype(jnp.int32)
    cap = ((counts_i + (B - 1)) // B) * B
    capf = cap.astype(jnp.float32)
    iu = lax.broadcasted_iota(jnp.int32, (E, E), 0)
    ju = lax.broadcasted_iota(jnp.int32, (E, E), 1)
    U8 = jnp.where(iu < ju, 1.0, 0.0)
    offs = lax.dot_general(capf, U8, (((1,), (0,)), ((), ())),
                           preferred_element_type=jnp.float32)
    ends = offs + capf

    for i in range(nblk):
        sl = slice(i * B, (i + 1) * B)
        rloc = lax.dot_general(Lb, selb[sl, :], (((1,), (0,)), ((), ())),
                               preferred_element_type=jnp.float32)
        dest = offs + pref[i:i + 1, :] + rloc
        d1 = jnp.sum(jnp.where(mask1[sl, :], dest, 0.0), axis=1, keepdims=True)
        d2 = jnp.sum(jnp.where(mask2[sl, :], dest, 0.0), axis=1, keepdims=True)
        d1_ref[sl, :] = d1.astype(jnp.int32)
        d2_ref[sl, :] = d2.astype(jnp.int32)

    lane8 = lax.broadcasted_iota(jnp.int32, (1, E), 1)
    lane128 = lax.broadcasted_iota(jnp.int32, (8, 128), 1)
    nb_acc = jnp.zeros((8, 128), jnp.float32)
    bo_acc = jnp.zeros((8, 128), jnp.float32)
    for e in range(E):
        nb_e = jnp.sum(jnp.where(lane8 == e, capf, 0.0)) * (1.0 / B)
        bo_e = jnp.sum(jnp.where(lane8 == e, offs, 0.0)) * (1.0 / B)
        nb_acc = nb_acc + jnp.where(lane128 == e, nb_e, 0.0)
        bo_acc = bo_acc + jnp.where(lane128 == e, bo_e, 0.0)
    nb_ref[...] = nb_acc.astype(jnp.int32)
    bo_ref[...] = bo_acc.astype(jnp.int32)


def _dispatch(x, Wr):
    return pl.pallas_call(
        _dispatch_kernel,
        in_specs=[pl.BlockSpec((T, H), lambda: (0, 0)),
                  pl.BlockSpec((E, H), lambda: (0, 0))],
        out_specs=[pl.BlockSpec((T, 1), lambda: (0, 0)),
                   pl.BlockSpec((T, 1), lambda: (0, 0)),
                   pl.BlockSpec((T, 1), lambda: (0, 0)),
                   pl.BlockSpec((T, 1), lambda: (0, 0)),
                   pl.BlockSpec((8, 128), lambda: (0, 0)),
                   pl.BlockSpec((8, 128), lambda: (0, 0))],
        out_shape=[jax.ShapeDtypeStruct((T, 1), jnp.int32),
                   jax.ShapeDtypeStruct((T, 1), jnp.int32),
                   jax.ShapeDtypeStruct((T, 1), jnp.float32),
                   jax.ShapeDtypeStruct((T, 1), jnp.float32),
                   jax.ShapeDtypeStruct((8, 128), jnp.int32),
                   jax.ShapeDtypeStruct((8, 128), jnp.int32)],
    )(x, Wr)


def _sc_scatter_body(x_hbm, d1_hbm, d2_hbm, xs_hbm,
                     idx1_v, idx2_v, rows_v, s1, s2):
    wid = lax.axis_index("s") * 2 + lax.axis_index("c")
    base = wid * TPW
    pltpu.sync_copy(d1_hbm.at[pl.ds(base, TPW)], idx1_v)
    pltpu.sync_copy(d2_hbm.at[pl.ds(base, TPW)], idx2_v)
    pltpu.sync_copy(x_hbm.at[pl.ds(base, TPW)], rows_v)
    c1 = pltpu.async_copy(rows_v, xs_hbm.at[idx1_v], s1)
    c2 = pltpu.async_copy(rows_v, xs_hbm.at[idx2_v], s2)
    c1.wait()
    c2.wait()


def _sc_scatter(x, d1f, d2f):
    fn = functools.partial(
        pl.kernel,
        mesh=plsc.VectorSubcoreMesh(core_axis_name="c", subcore_axis_name="s"),
        out_type=jax.ShapeDtypeStruct((SPAD, H), jnp.float32),
        scratch_types=[pltpu.VMEM((TPW,), jnp.int32),
                       pltpu.VMEM((TPW,), jnp.int32),
                       pltpu.VMEM((TPW, H), jnp.float32),
                       pltpu.SemaphoreType.DMA,
                       pltpu.SemaphoreType.DMA],
    )(_sc_scatter_body)
    return fn(x, d1f, d2f)


NCG = 4
ICW = I // NCG
NCD = 3
HCW = H // NCD


def _moe_blocks_kernel(nbe_ref, boff_ref, xs_hbm, *refs):
    wg_refs = refs[0:NCG]
    wu_refs = refs[NCG:2 * NCG]
    wd_refs = refs[2 * NCG:2 * NCG + NCD]
    ys_hbm, xbuf, ybuf, ht, sin, sout = refs[2 * NCG + NCD:]
    e = pl.program_id(0)
    n = nbe_ref[e]
    base = boff_ref[e]

    def rd(j, slot):
        return pltpu.make_async_copy(
            xs_hbm.at[pl.ds((base + j) * B, B), :], xbuf.at[slot],
            sin.at[slot])

    def wr(j, slot):
        return pltpu.make_async_copy(
            ybuf.at[slot], ys_hbm.at[pl.ds((base + j) * B, B), :],
            sout.at[slot])

    @pl.when(n > 0)
    def _go():
        rd(0, 0).start()

        def body(j, carry):
            slot = lax.rem(j, 2)
            nslot = lax.rem(j + 1, 2)

            @pl.when(j + 1 < n)
            def _prefetch():
                rd(j + 1, nslot).start()

            rd(j, slot).wait()
            xb = xbuf[slot]
            for c in range(NCG):
                g = lax.dot_general(wg_refs[c][0], xb, (((1,), (1,)), ((), ())),
                                    preferred_element_type=jnp.float32)
                u = lax.dot_general(wu_refs[c][0], xb, (((1,), (1,)), ((), ())),
                                    preferred_element_type=jnp.float32)
                ht[pl.ds(c * ICW, ICW), :] = g * jax.nn.sigmoid(g) * u

            @pl.when(j >= 2)
            def _drain_prev():
                wr(j - 2, slot).wait()

            hts = ht[...]
            for c in range(NCD):
                yc = lax.dot_general(hts, wd_refs[c][0], (((0,), (1,)), ((), ())),
                                     preferred_element_type=jnp.float32)
                ybuf[slot, :, pl.ds(c * HCW, HCW)] = yc

            wr(j, slot).start()
            return carry

        lax.fori_loop(0, n, body, 0)

        @pl.when(n >= 2)
        def _drain2():
            wr(n - 2, lax.rem(n, 2)).wait()

        wr(n - 1, lax.rem(n + 1, 2)).wait()


def _moe_blocks(nbe, boff, xs, Wg, Wu, Wd):
    def cmap(c):
        return lambda e, nbe, boff: (e, c, 0)

    grid_spec = pltpu.PrefetchScalarGridSpec(
        num_scalar_prefetch=2,
        grid=(E,),
        in_specs=(
            [pl.BlockSpec(memory_space=pl.ANY)]
            + [pl.BlockSpec((1, ICW, H), cmap(c)) for c in range(NCG)]
            + [pl.BlockSpec((1, ICW, H), cmap(c)) for c in range(NCG)]
            + [pl.BlockSpec((1, HCW, I), cmap(c)) for c in range(NCD)]
        ),
        out_specs=pl.BlockSpec(memory_space=pl.ANY),
        scratch_shapes=[pltpu.VMEM((2, B, H), jnp.float32),
                        pltpu.VMEM((2, B, H), jnp.float32),
                        pltpu.VMEM((I, B), jnp.float32),
                        pltpu.SemaphoreType.DMA((2,)),
                        pltpu.SemaphoreType.DMA((2,))],
    )
    args = ([xs] + [Wg] * NCG + [Wu] * NCG + [Wd] * NCD)
    return pl.pallas_call(
        _moe_blocks_kernel,
        grid_spec=grid_spec,
        out_shape=jax.ShapeDtypeStruct((SPAD, H), jnp.float32),
    )(nbe, boff, *args)


def _sc_gather_body(ys_hbm, d1_hbm, d2_hbm, y1_hbm, y2_hbm,
                    idx1_v, idx2_v, buf1, buf2, s1, s2):
    wid = lax.axis_index("s") * 2 + lax.axis_index("c")
    base = wid * TPW
    pltpu.sync_copy(d1_hbm.at[pl.ds(base, TPW)], idx1_v)
    pltpu.sync_copy(d2_hbm.at[pl.ds(base, TPW)], idx2_v)
    g1 = pltpu.async_copy(ys_hbm.at[idx1_v], buf1, s1)
    g2 = pltpu.async_copy(ys_hbm.at[idx2_v], buf2, s2)
    g1.wait()
    g2.wait()
    pltpu.sync_copy(buf1, y1_hbm.at[pl.ds(base, TPW)])
    pltpu.sync_copy(buf2, y2_hbm.at[pl.ds(base, TPW)])


def _sc_gather(ys, d1f, d2f):
    fn = functools.partial(
        pl.kernel,
        mesh=plsc.VectorSubcoreMesh(core_axis_name="c", subcore_axis_name="s"),
        out_type=[jax.ShapeDtypeStruct((T, H), jnp.float32),
                  jax.ShapeDtypeStruct((T, H), jnp.float32)],
        scratch_types=[pltpu.VMEM((TPW,), jnp.int32),
                       pltpu.VMEM((TPW,), jnp.int32),
                       pltpu.VMEM((TPW, H), jnp.float32),
                       pltpu.VMEM((TPW, H), jnp.float32),
                       pltpu.SemaphoreType.DMA,
                       pltpu.SemaphoreType.DMA],
    )(_sc_gather_body)
    return fn(ys, d1f, d2f)


def _wsum_kernel(y1_ref, y2_ref, w1_ref, w2_ref, out_ref):
    out_ref[...] = w1_ref[...] * y1_ref[...] + w2_ref[...] * y2_ref[...]


def _wsum(y1, y2, w1, w2):
    tb = 256
    return pl.pallas_call(
        _wsum_kernel,
        grid=(T // tb,),
        in_specs=[pl.BlockSpec((tb, H), lambda i: (i, 0)),
                  pl.BlockSpec((tb, H), lambda i: (i, 0)),
                  pl.BlockSpec((tb, 1), lambda i: (i, 0)),
                  pl.BlockSpec((tb, 1), lambda i: (i, 0))],
        out_specs=pl.BlockSpec((tb, H), lambda i: (i, 0)),
        out_shape=jax.ShapeDtypeStruct((T, H), jnp.float32),
    )(y1, y2, w1, w2)


@jax.jit
def kernel(x, Wr, Wg, Wu, Wd):
    d1, d2, w1, w2, nb, bo = _dispatch(x, Wr)
    d1f = d1.reshape(T)
    d2f = d2.reshape(T)
    nbe = nb[0, :E]
    boff = bo[0, :E]
    xs = _sc_scatter(x, d1f, d2f)
    ys = _moe_blocks(nbe, boff, xs, Wg, Wu, Wd)
    y1, y2 = _sc_gather(ys, d1f, d2f)
    return _wsum(y1, y2, w1, w2)


# speedup vs baseline: 1.1625x; 1.1625x over previous
import functools

import jax
import jax.numpy as jnp
from jax import lax
from jax.experimental import pallas as pl
from jax.experimental.pallas import tpu as pltpu
from jax.experimental.pallas import tpu_sc as plsc

E = 8
H = 768
I = 2112
T = 2048
B = 256
NB = T * 2 // B + E
SPAD = NB * B
NW = 32
TPW = T // NW


def _dispatch_kernel(x_ref, wr_ref, d1_ref, d2_ref, w1_ref, w2_ref, be_ref):
    x = x_ref[...]
    logits = lax.dot_general(x, wr_ref[...], (((1,), (1,)), ((), ())),
                             preferred_element_type=jnp.float32)
    lane = lax.broadcasted_iota(jnp.int32, (T, E), 1)
    m1 = jnp.max(logits, axis=1, keepdims=True)
    first1 = jnp.min(jnp.where(logits >= m1, lane, E), axis=1, keepdims=True)
    mask1 = lane == first1
    l2 = jnp.where(mask1, -jnp.inf, logits)
    m2 = jnp.max(l2, axis=1, keepdims=True)
    first2 = jnp.min(jnp.where(l2 >= m2, lane, E), axis=1, keepdims=True)
    mask2 = lane == first2
    s = jax.nn.sigmoid(m1 - m2)
    w1_ref[...] = s
    w2_ref[...] = 1.0 - s

    sel = jnp.where(mask1 | mask2, 1.0, 0.0)
    selb = sel.astype(jnp.bfloat16)

    it = lax.broadcasted_iota(jnp.int32, (B, B), 0)
    ic = lax.broadcasted_iota(jnp.int32, (B, B), 1)
    Lb = jnp.where(it > ic, 1.0, 0.0).astype(jnp.bfloat16)

    nblk = T // B
    rows = [jnp.sum(sel[i * B:(i + 1) * B, :], axis=0, keepdims=True)
            for i in range(nblk)]
    bs = jnp.concatenate(rows, axis=0)
    itb = lax.broadcasted_iota(jnp.int32, (nblk, nblk), 0)
    icb = lax.broadcasted_iota(jnp.int32, (nblk, nblk), 1)
    L8 = jnp.where(itb > icb, 1.0, 0.0)
    pref = lax.dot_general(L8, bs, (((1,), (0,)), ((), ())),
                           preferred_element_type=jnp.float32)
    counts = pref[nblk - 1:nblk, :] + bs[nblk - 1:nblk, :]
    counts_i = counts.astype(jnp.int32)
    cap = ((counts_i + (B - 1)) // B) * B
    capf = cap.astype(jnp.float32)
    iu = lax.broadcasted_iota(jnp.int32, (E, E), 0)
    ju = lax.broadcasted_iota(jnp.int32, (E, E), 1)
    U8 = jnp.where(iu < ju, 1.0, 0.0)
    offs = lax.dot_general(capf, U8, (((1,), (0,)), ((), ())),
                           preferred_element_type=jnp.float32)
    ends = offs + capf

    for i in range(nblk):
        sl = slice(i * B, (i + 1) * B)
        rloc = lax.dot_general(Lb, selb[sl, :], (((1,), (0,)), ((), ())),
                               preferred_element_type=jnp.float32)
        dest = offs + pref[i:i + 1, :] + rloc
        d1 = jnp.sum(jnp.where(mask1[sl, :], dest, 0.0), axis=1, keepdims=True)
        d2 = jnp.sum(jnp.where(mask2[sl, :], dest, 0.0), axis=1, keepdims=True)
        d1_ref[sl, :] = d1.astype(jnp.int32)
        d2_ref[sl, :] = d2.astype(jnp.int32)

    bi = lax.broadcasted_iota(jnp.int32, (8, 128), 0)
    bj = lax.broadcasted_iota(jnp.int32, (8, 128), 1)
    bstart = ((bi * 128 + bj) * B).astype(jnp.float32)
    acc = jnp.zeros((8, 128), jnp.int32)
    lane8 = lax.broadcasted_iota(jnp.int32, (1, E), 1)
    for e in range(E):
        end_e = jnp.sum(jnp.where(lane8 == e, ends, 0.0))
        acc = acc + jnp.where(bstart >= end_e, 1, 0)
    be_ref[...] = acc


def _dispatch(x, Wr):
    return pl.pallas_call(
        _dispatch_kernel,
        in_specs=[pl.BlockSpec((T, H), lambda: (0, 0)),
                  pl.BlockSpec((E, H), lambda: (0, 0))],
        out_specs=[pl.BlockSpec((T, 1), lambda: (0, 0)),
                   pl.BlockSpec((T, 1), lambda: (0, 0)),
                   pl.BlockSpec((T, 1), lambda: (0, 0)),
                   pl.BlockSpec((T, 1), lambda: (0, 0)),
                   pl.BlockSpec((8, 128), lambda: (0, 0))],
        out_shape=[jax.ShapeDtypeStruct((T, 1), jnp.int32),
                   jax.ShapeDtypeStruct((T, 1), jnp.int32),
                   jax.ShapeDtypeStruct((T, 1), jnp.float32),
                   jax.ShapeDtypeStruct((T, 1), jnp.float32),
                   jax.ShapeDtypeStruct((8, 128), jnp.int32)],
    )(x, Wr)


def _sc_scatter_body(x_hbm, d1_hbm, d2_hbm, xs_hbm,
                     idx1_v, idx2_v, rows_v, s1, s2):
    wid = lax.axis_index("s") * 2 + lax.axis_index("c")
    base = wid * TPW
    pltpu.sync_copy(d1_hbm.at[pl.ds(base, TPW)], idx1_v)
    pltpu.sync_copy(d2_hbm.at[pl.ds(base, TPW)], idx2_v)
    pltpu.sync_copy(x_hbm.at[pl.ds(base, TPW)], rows_v)
    c1 = pltpu.async_copy(rows_v, xs_hbm.at[idx1_v], s1)
    c2 = pltpu.async_copy(rows_v, xs_hbm.at[idx2_v], s2)
    c1.wait()
    c2.wait()


def _sc_scatter(x, d1f, d2f):
    fn = functools.partial(
        pl.kernel,
        mesh=plsc.VectorSubcoreMesh(core_axis_name="c", subcore_axis_name="s"),
        out_type=jax.ShapeDtypeStruct((SPAD, H), jnp.float32),
        scratch_types=[pltpu.VMEM((TPW,), jnp.int32),
                       pltpu.VMEM((TPW,), jnp.int32),
                       pltpu.VMEM((TPW, H), jnp.float32),
                       pltpu.SemaphoreType.DMA,
                       pltpu.SemaphoreType.DMA],
    )(_sc_scatter_body)
    return fn(x, d1f, d2f)


def _moe_blocks_kernel(be_ref, xs_ref, wg_ref, wu_ref, wd_ref, ys_ref):
    b = pl.program_id(0)

    @pl.when(be_ref[b] < E)
    def _compute():
        xb = xs_ref[...]
        gate = lax.dot_general(xb, wg_ref[0], (((1,), (1,)), ((), ())),
                               preferred_element_type=jnp.float32)
        up = lax.dot_general(xb, wu_ref[0], (((1,), (1,)), ((), ())),
                             preferred_element_type=jnp.float32)
        h = gate * jax.nn.sigmoid(gate) * up
        ys_ref[...] = lax.dot_general(h, wd_ref[0], (((1,), (1,)), ((), ())),
                                      preferred_element_type=jnp.float32)


def _moe_blocks(bes, xs, Wg, Wu, Wd):
    def wmap(b, be):
        return (jnp.minimum(be[b], E - 1), 0, 0)

    grid_spec = pltpu.PrefetchScalarGridSpec(
        num_scalar_prefetch=1,
        grid=(NB,),
        in_specs=[
            pl.BlockSpec((B, H), lambda b, be: (b, 0)),
            pl.BlockSpec((1, I, H), wmap),
            pl.BlockSpec((1, I, H), wmap),
            pl.BlockSpec((1, H, I), wmap),
        ],
        out_specs=pl.BlockSpec((B, H), lambda b, be: (b, 0)),
    )
    return pl.pallas_call(
        _moe_blocks_kernel,
        grid_spec=grid_spec,
        out_shape=jax.ShapeDtypeStruct((SPAD, H), jnp.float32),
    )(bes, xs, Wg, Wu, Wd)


def _sc_gather_body(ys_hbm, d1_hbm, d2_hbm, y1_hbm, y2_hbm,
                    idx1_v, idx2_v, buf1, buf2, s1, s2):
    wid = lax.axis_index("s") * 2 + lax.axis_index("c")
    base = wid * TPW
    pltpu.sync_copy(d1_hbm.at[pl.ds(base, TPW)], idx1_v)
    pltpu.sync_copy(d2_hbm.at[pl.ds(base, TPW)], idx2_v)
    g1 = pltpu.async_copy(ys_hbm.at[idx1_v], buf1, s1)
    g2 = pltpu.async_copy(ys_hbm.at[idx2_v], buf2, s2)
    g1.wait()
    g2.wait()
    pltpu.sync_copy(buf1, y1_hbm.at[pl.ds(base, TPW)])
    pltpu.sync_copy(buf2, y2_hbm.at[pl.ds(base, TPW)])


def _sc_gather(ys, d1f, d2f):
    fn = functools.partial(
        pl.kernel,
        mesh=plsc.VectorSubcoreMesh(core_axis_name="c", subcore_axis_name="s"),
        out_type=[jax.ShapeDtypeStruct((T, H), jnp.float32),
                  jax.ShapeDtypeStruct((T, H), jnp.float32)],
        scratch_types=[pltpu.VMEM((TPW,), jnp.int32),
                       pltpu.VMEM((TPW,), jnp.int32),
                       pltpu.VMEM((TPW, H), jnp.float32),
                       pltpu.VMEM((TPW, H), jnp.float32),
                       pltpu.SemaphoreType.DMA,
                       pltpu.SemaphoreType.DMA],
    )(_sc_gather_body)
    return fn(ys, d1f, d2f)


def _wsum_kernel(y1_ref, y2_ref, w1_ref, w2_ref, out_ref):
    out_ref[...] = w1_ref[...] * y1_ref[...] + w2_ref[...] * y2_ref[...]


def _wsum(y1, y2, w1, w2):
    tb = 256
    return pl.pallas_call(
        _wsum_kernel,
        grid=(T // tb,),
        in_specs=[pl.BlockSpec((tb, H), lambda i: (i, 0)),
                  pl.BlockSpec((tb, H), lambda i: (i, 0)),
                  pl.BlockSpec((tb, 1), lambda i: (i, 0)),
                  pl.BlockSpec((tb, 1), lambda i: (i, 0))],
        out_specs=pl.BlockSpec((tb, H), lambda i: (i, 0)),
        out_shape=jax.ShapeDtypeStruct((T, H), jnp.float32),
    )(y1, y2, w1, w2)


@jax.jit
def kernel(x, Wr, Wg, Wu, Wd):
    d1, d2, w1, w2, be = _dispatch(x, Wr)
    d1f = d1.reshape(T)
    d2f = d2.reshape(T)
    bes = be.reshape(-1)[:NB]
    xs = _sc_scatter(x, d1f, d2f)
    ys = _moe_blocks(bes, xs, Wg, Wu, Wd)
    y1, y2 = _sc_gather(ys, d1f, d2f)
    return _wsum(y1, y2, w1, w2)
